# Initial kernel scaffold; baseline (speedup 1.0000x reference)
import jax
import jax.numpy as jnp
from jax.experimental import pallas as pl


def kernel(table, x):
    raise NotImplementedError("write your pallas kernel here")


# speedup vs baseline: 59.0575x; 59.0575x over previous
import jax
import jax.numpy as jnp
from jax.experimental import pallas as pl
from jax.experimental.pallas import tpu as pltpu

_ROWS_PER_STEP = 512


def _gather_kernel(idx_ref, tab_ref, out_ref, *, rows):
    for mi in range(rows):
        v = idx_ref[0, 0, mi]
        out_ref[mi, :] = tab_ref[v, 0, :]


def kernel(table, x):
    V, D = table.shape
    B, F = x.shape
    N = B * F

    R = min(_ROWS_PER_STEP, N)
    n_steps = pl.cdiv(N, R)
    N_pad = n_steps * R

    flat_idx = x.reshape(-1).astype(jnp.int32)
    if N_pad != N:
        flat_idx = jnp.pad(flat_idx, (0, N_pad - N))
    idx3 = flat_idx.reshape(n_steps, 1, R)

    tab3 = table.reshape(V, 1, D)

    out = pl.pallas_call(
        lambda idx_ref, tab_ref, out_ref: _gather_kernel(
            idx_ref, tab_ref, out_ref, rows=R),
        out_shape=jax.ShapeDtypeStruct((N_pad, D), table.dtype),
        grid=(n_steps,),
        in_specs=[
            pl.BlockSpec((1, 1, R), lambda i: (i, 0, 0),
                         memory_space=pltpu.SMEM),
            pl.BlockSpec((V, 1, D), lambda i: (0, 0, 0)),
        ],
        out_specs=pl.BlockSpec((R, D), lambda i: (i, 0)),
        compiler_params=pltpu.CompilerParams(
            dimension_semantics=("parallel",),
            vmem_limit_bytes=48 * 1024 * 1024),
        cost_estimate=pl.CostEstimate(
            flops=0,
            transcendentals=0,
            bytes_accessed=N_pad * 4 + V * D * 4 + N_pad * D * 4),
    )(idx3, tab3)

    if N_pad != N:
        out = out[:N]
    return out.reshape(B, F, D)


# speedup vs baseline: 93.1959x; 1.5781x over previous
import jax
import jax.numpy as jnp
from jax.experimental import pallas as pl
from jax.experimental.pallas import tpu as pltpu

_ROWS_PER_STEP = 2048


def _gather_kernel(idx_ref, tab_ref, out_ref, *, rows):
    for mi in range(rows):
        v = idx_ref[0, 0, mi]
        out_ref[mi, :] = tab_ref[v, 0, :]


def kernel(table, x):
    V, D = table.shape
    B, F = x.shape
    N = B * F

    R = min(_ROWS_PER_STEP, N)
    n_steps = pl.cdiv(N, R)
    N_pad = n_steps * R

    flat_idx = x.reshape(-1).astype(jnp.int32)
    if N_pad != N:
        flat_idx = jnp.pad(flat_idx, (0, N_pad - N))
    idx3 = flat_idx.reshape(n_steps, 1, R)

    tab3 = table.reshape(V, 1, D)

    out = pl.pallas_call(
        lambda idx_ref, tab_ref, out_ref: _gather_kernel(
            idx_ref, tab_ref, out_ref, rows=R),
        out_shape=jax.ShapeDtypeStruct((N_pad, D), table.dtype),
        grid=(n_steps,),
        in_specs=[
            pl.BlockSpec((1, 1, R), lambda i: (i, 0, 0),
                         memory_space=pltpu.SMEM),
            pl.BlockSpec((V, 1, D), lambda i: (0, 0, 0)),
        ],
        out_specs=pl.BlockSpec((R, D), lambda i: (i, 0)),
        compiler_params=pltpu.CompilerParams(
            dimension_semantics=("parallel",),
            vmem_limit_bytes=48 * 1024 * 1024),
        cost_estimate=pl.CostEstimate(
            flops=0,
            transcendentals=0,
            bytes_accessed=N_pad * 4 + V * D * 4 + N_pad * D * 4),
    )(idx3, tab3)

    if N_pad != N:
        out = out[:N]
    return out.reshape(B, F, D)
